# baseline (speedup 1.0000x reference)
import jax
import jax.numpy as jnp
from jax.experimental import pallas as pl


_S = 2
_BN = 4096


def _body(x_ref, wa_ref, wb_ref, b_ref, o_ref):
    dims = (((1,), (1,)), ((), ()))
    o_ref[:, 0, :] = jax.lax.dot_general(
        x_ref[...], wa_ref[0], dims, preferred_element_type=jnp.float32
    ) + b_ref[0, 0, :]
    o_ref[:, 1, :] = jax.lax.dot_general(
        x_ref[...], wb_ref[0], dims, preferred_element_type=jnp.float32
    ) + b_ref[0, 1, :]


def kernel(x, tgt, table_w0, table_b0):
    del tgt
    B, I, H = x.shape
    N = table_w0.shape[0]
    NS = N // _S
    x2 = x.reshape(B * I, H)
    w3 = table_w0.reshape(_S, NS, H)
    b3 = table_b0.reshape(1, _S, NS)
    out = pl.pallas_call(
        _body,
        grid=(pl.cdiv(NS, _BN),),
        in_specs=[
            pl.BlockSpec((B * I, H), lambda i: (0, 0)),
            pl.BlockSpec((1, _BN, H), lambda i: (0, i, 0)),
            pl.BlockSpec((1, _BN, H), lambda i: (1, i, 0)),
            pl.BlockSpec((1, _S, _BN), lambda i: (0, 0, i)),
        ],
        out_specs=pl.BlockSpec((B * I, _S, _BN), lambda i: (0, 0, i)),
        out_shape=jax.ShapeDtypeStruct((B * I, _S, NS), jnp.float32),
    )(x2, w3, w3, b3)
    return out.reshape(B, I, N)


# speedup vs baseline: 1.0475x; 1.0475x over previous
import jax
import jax.numpy as jnp
from jax.experimental import pallas as pl


_S = 2
_BN = 2048


def _body(x_ref, wa_ref, wb_ref, b_ref, o_ref):
    dims = (((1,), (1,)), ((), ()))
    o_ref[:, 0, :] = jax.lax.dot_general(
        x_ref[...], wa_ref[0], dims, preferred_element_type=jnp.float32
    ) + b_ref[0, 0, :]
    o_ref[:, 1, :] = jax.lax.dot_general(
        x_ref[...], wb_ref[0], dims, preferred_element_type=jnp.float32
    ) + b_ref[0, 1, :]


def kernel(x, tgt, table_w0, table_b0):
    del tgt
    B, I, H = x.shape
    N = table_w0.shape[0]
    NS = N // _S
    x2 = x.reshape(B * I, H)
    w3 = table_w0.reshape(_S, NS, H)
    b3 = table_b0.reshape(1, _S, NS)
    out = pl.pallas_call(
        _body,
        grid=(pl.cdiv(NS, _BN),),
        in_specs=[
            pl.BlockSpec((B * I, H), lambda i: (0, 0)),
            pl.BlockSpec((1, _BN, H), lambda i: (0, i, 0)),
            pl.BlockSpec((1, _BN, H), lambda i: (1, i, 0)),
            pl.BlockSpec((1, _S, _BN), lambda i: (0, 0, i)),
        ],
        out_specs=pl.BlockSpec((B * I, _S, _BN), lambda i: (0, 0, i)),
        out_shape=jax.ShapeDtypeStruct((B * I, _S, NS), jnp.float32),
    )(x2, w3, w3, b3)
    return out.reshape(B, I, N)
